# baseline (speedup 1.0000x reference)
import functools

import jax
import jax.numpy as jnp
from jax import lax
from jax.experimental import pallas as pl
from jax.experimental.pallas import tpu as pltpu
from jax.experimental.pallas import tpu_sc as plsc

N = 8192
NW = 32
QW = N // NW
CH = N // 16
R = 128

_INF = float("inf")


def _sc_knn_make():
    mesh = plsc.VectorSubcoreMesh(core_axis_name="c", subcore_axis_name="s")

    @functools.partial(
        pl.kernel, mesh=mesh,
        out_type=[jax.ShapeDtypeStruct((N,), jnp.float32)] * 3,
        scratch_types=[
            pltpu.VMEM((N,), jnp.float32),
            pltpu.VMEM((N,), jnp.float32),
            pltpu.VMEM((N,), jnp.float32),
            pltpu.VMEM((N,), jnp.float32),
            pltpu.VMEM((QW,), jnp.float32),
            pltpu.VMEM((QW,), jnp.float32),
            pltpu.VMEM((QW,), jnp.float32),
        ],
    )
    def knn(xs_h, ys_h, zs_h, cb_h,
            oj1, oj2, oj3,
            xs, ys, zs, cb, j1v, j2v, j3v):
        wid = lax.axis_index("s") * 2 + lax.axis_index("c")
        base = wid * QW
        pltpu.sync_copy(xs_h, xs)
        pltpu.sync_copy(ys_h, ys)
        pltpu.sync_copy(zs_h, zs)
        pltpu.sync_copy(cb_h, cb)

        def per_group(g, carry):
            qoff = base + g * 16
            qx = xs[pl.ds(qoff, 16)]
            qy = ys[pl.ds(qoff, 16)]
            qz = zs[pl.ds(qoff, 16)]
            qx2 = qx + qx
            qy2 = qy + qy
            qz2 = qz + qz

            def per_chunk(c, st):
                m1, m2, m3, i1, i2, i3 = st
                off = c * 16
                xc = xs[pl.ds(off, 16)]
                yc = ys[pl.ds(off, 16)]
                zc = zs[pl.ds(off, 16)]
                cbc = cb[pl.ds(off, 16)]
                for k in range(16):
                    d = cbc[k] - (qx2 * xc[k] + qy2 * yc[k] + qz2 * zc[k])
                    iv = jnp.full((16,), off + k, jnp.int32)
                    c1 = d < m1
                    c2 = d < m2
                    c3 = d < m3
                    m3 = jnp.where(c2, m2, jnp.where(c3, d, m3))
                    i3 = jnp.where(c2, i2, jnp.where(c3, iv, i3))
                    m2 = jnp.where(c1, m1, jnp.where(c2, d, m2))
                    i2 = jnp.where(c1, i1, jnp.where(c2, iv, i2))
                    m1 = jnp.where(c1, d, m1)
                    i1 = jnp.where(c1, iv, i1)
                return m1, m2, m3, i1, i2, i3

            init = (jnp.full((16,), _INF, jnp.float32),
                    jnp.full((16,), _INF, jnp.float32),
                    jnp.full((16,), _INF, jnp.float32),
                    jnp.zeros((16,), jnp.int32), jnp.zeros((16,), jnp.int32),
                    jnp.zeros((16,), jnp.int32))
            m1, m2, m3, i1, i2, i3 = lax.fori_loop(0, CH, per_chunk, init)

            s = g * 16
            j1v[pl.ds(s, 16)] = i1.astype(jnp.float32)
            j2v[pl.ds(s, 16)] = i2.astype(jnp.float32)
            j3v[pl.ds(s, 16)] = i3.astype(jnp.float32)
            return carry

        lax.fori_loop(0, QW // 16, per_group, 0)

        pltpu.sync_copy(j1v, oj1.at[pl.ds(base, QW)])
        pltpu.sync_copy(j2v, oj2.at[pl.ds(base, QW)])
        pltpu.sync_copy(j3v, oj3.at[pl.ds(base, QW)])

    return knn


def _tc_body(j1_ref, j2_ref, j3_ref, anum_ref, c0_ref, c1_ref, c2_ref,
             pos_full_ref, o0_ref, o1_ref, o2_ref):
    i = pl.program_id(0)
    xr = pos_full_ref[pl.ds(i * R, R), :]
    pos_tab = pos_full_ref[...]
    iota = jax.lax.broadcasted_iota(jnp.int32, (R, N), 1).astype(jnp.float32)

    is_heavy = anum_ref[...] != 1
    ja = jnp.where(is_heavy, j2_ref[...], j1_ref[...])
    jb = jnp.where(is_heavy, j3_ref[...], j2_ref[...])
    ga = jnp.dot(jnp.where(iota == ja, 1.0, 0.0), pos_tab,
                 preferred_element_type=jnp.float32)
    gb = jnp.dot(jnp.where(iota == jb, 1.0, 0.0), pos_tab,
                 preferred_element_type=jnp.float32)

    rel_a = ga - xr
    rel_b = gb - xr
    la = jnp.sqrt(jnp.sum(rel_a * rel_a, axis=1, keepdims=True))
    lb = jnp.sqrt(jnp.sum(rel_b * rel_b, axis=1, keepdims=True))
    take_a = (la - lb) <= 0.0
    p1 = jnp.where(take_a, rel_a, rel_b)
    p2 = jnp.where(take_a, rel_b, rel_a)

    e1 = p1 / jnp.sqrt(jnp.sum(p1 * p1, axis=1, keepdims=True))
    proj = jnp.sum(p2 * e1, axis=1, keepdims=True)
    u2 = p2 - proj * e1
    e2 = u2 / jnp.sqrt(jnp.sum(u2 * u2, axis=1, keepdims=True))
    e3 = jnp.concatenate([
        e1[:, 1:2] * e2[:, 2:3] - e1[:, 2:3] * e2[:, 1:2],
        e1[:, 2:3] * e2[:, 0:1] - e1[:, 0:1] * e2[:, 2:3],
        e1[:, 0:1] * e2[:, 1:2] - e1[:, 1:2] * e2[:, 0:1],
    ], axis=1)
    c0 = c0_ref[...]
    c1 = c1_ref[...]
    c2 = c2_ref[...]
    for e, o_ref in ((e1, o0_ref), (e2, o1_ref), (e3, o2_ref)):
        o_ref[...] = e[:, 0:1] * c0 + e[:, 1:2] * c1 + e[:, 2:3] * c2


@jax.jit
def kernel(coeffs, pos, atomic_numbers):
    heavy = atomic_numbers != 1
    colbase = (jnp.sum(pos * pos, axis=1)
               + jnp.where(heavy, 0.0, jnp.inf)).astype(jnp.float32)
    xs = pos[:, 0]
    ys = pos[:, 1]
    zs = pos[:, 2]

    j1, j2, j3 = _sc_knn_make()(xs, ys, zs, colbase)

    anum = atomic_numbers[:, None]
    c0 = coeffs[:, :, 0]
    c1 = coeffs[:, :, 1]
    c2 = coeffs[:, :, 2]
    full = lambda *dims: pl.BlockSpec(dims, lambda i: (0,) * len(dims))
    rows = lambda *dims: pl.BlockSpec((R,) + dims, lambda i: (i,) + (0,) * len(dims))
    o0, o1, o2 = pl.pallas_call(
        _tc_body,
        grid=(N // R,),
        in_specs=[rows(1), rows(1), rows(1), rows(1),
                  rows(64), rows(64), rows(64), full(N, 3)],
        out_specs=[rows(64), rows(64), rows(64)],
        out_shape=[jax.ShapeDtypeStruct((N, 64), jnp.float32)] * 3,
    )(j1[:, None], j2[:, None], j3[:, None], anum, c0, c1, c2, pos)
    return jnp.stack([o0, o1, o2], axis=-1)


# speedup vs baseline: 2.6359x; 2.6359x over previous
import functools

import jax
import jax.numpy as jnp
from jax import lax
from jax.experimental import pallas as pl
from jax.experimental.pallas import tpu as pltpu
from jax.experimental.pallas import tpu_sc as plsc

N = 8192
R = 128
S_TC = 5632
N_SC = N - S_TC
NW = 32
QW = N_SC // NW
CH = N // 16

_INF = float("inf")


def _tc_main_body(posT2_ref, colbase_ref, anum_ref, c0_ref, c1_ref, c2_ref,
                  pos_full_ref, o0_ref, o1_ref, o2_ref):
    i = pl.program_id(0)
    xr = pos_full_ref[pl.ds(i * R, R), :]

    xr0 = xr[:, 0:1]
    xr1 = xr[:, 1:2]
    xr2 = xr[:, 2:3]
    r2r = xr0 * xr0 + xr1 * xr1 + xr2 * xr2

    dot2 = jnp.dot(xr, posT2_ref[...], preferred_element_type=jnp.float32)
    dm = (r2r + colbase_ref[...]) - dot2

    iota = jax.lax.broadcasted_iota(jnp.int32, (R, N), 1).astype(jnp.float32)
    pos_tab = pos_full_ref[...]

    gs = []
    for t in range(3):
        m = jnp.min(dm, axis=1, keepdims=True)
        idx = jnp.min(jnp.where(dm == m, iota, jnp.float32(2 * N)),
                      axis=1, keepdims=True)
        oh = iota == idx
        g = jnp.dot(jnp.where(oh, 1.0, 0.0), pos_tab,
                    preferred_element_type=jnp.float32)
        gs.append(g)
        if t < 2:
            dm = jnp.where(oh, jnp.inf, dm)

    is_heavy = anum_ref[...] != 1
    ga = jnp.where(is_heavy, gs[1], gs[0])
    gb = jnp.where(is_heavy, gs[2], gs[1])
    _frames_rotate(ga - xr, gb - xr, c0_ref, c1_ref, c2_ref,
                   o0_ref, o1_ref, o2_ref)


def _frames_rotate(rel_a, rel_b, c0_ref, c1_ref, c2_ref,
                   o0_ref, o1_ref, o2_ref):
    la = jnp.sqrt(jnp.sum(rel_a * rel_a, axis=1, keepdims=True))
    lb = jnp.sqrt(jnp.sum(rel_b * rel_b, axis=1, keepdims=True))
    take_a = (la - lb) <= 0.0
    p1 = jnp.where(take_a, rel_a, rel_b)
    p2 = jnp.where(take_a, rel_b, rel_a)

    e1 = p1 / jnp.sqrt(jnp.sum(p1 * p1, axis=1, keepdims=True))
    proj = jnp.sum(p2 * e1, axis=1, keepdims=True)
    u2 = p2 - proj * e1
    e2 = u2 / jnp.sqrt(jnp.sum(u2 * u2, axis=1, keepdims=True))
    e3 = jnp.concatenate([
        e1[:, 1:2] * e2[:, 2:3] - e1[:, 2:3] * e2[:, 1:2],
        e1[:, 2:3] * e2[:, 0:1] - e1[:, 0:1] * e2[:, 2:3],
        e1[:, 0:1] * e2[:, 1:2] - e1[:, 1:2] * e2[:, 0:1],
    ], axis=1)
    c0 = c0_ref[...]
    c1 = c1_ref[...]
    c2 = c2_ref[...]
    for e, o_ref in ((e1, o0_ref), (e2, o1_ref), (e3, o2_ref)):
        o_ref[...] = e[:, 0:1] * c0 + e[:, 1:2] * c1 + e[:, 2:3] * c2


def _sc_knn_make():
    mesh = plsc.VectorSubcoreMesh(core_axis_name="c", subcore_axis_name="s")

    @functools.partial(
        pl.kernel, mesh=mesh,
        out_type=[jax.ShapeDtypeStruct((N_SC,), jnp.float32)] * 3,
        scratch_types=[
            pltpu.VMEM((N,), jnp.float32),
            pltpu.VMEM((N,), jnp.float32),
            pltpu.VMEM((N,), jnp.float32),
            pltpu.VMEM((N,), jnp.float32),
            pltpu.VMEM((QW,), jnp.float32),
            pltpu.VMEM((QW,), jnp.float32),
            pltpu.VMEM((QW,), jnp.float32),
        ],
    )
    def knn(xs_h, ys_h, zs_h, cb_h,
            oj1, oj2, oj3,
            xs, ys, zs, cb, j1v, j2v, j3v):
        wid = lax.axis_index("s") * 2 + lax.axis_index("c")
        base = wid * QW
        pltpu.sync_copy(xs_h, xs)
        pltpu.sync_copy(ys_h, ys)
        pltpu.sync_copy(zs_h, zs)
        pltpu.sync_copy(cb_h, cb)

        def per_group(g, carry):
            qoff = S_TC + base + g * 16
            qx = xs[pl.ds(qoff, 16)]
            qy = ys[pl.ds(qoff, 16)]
            qz = zs[pl.ds(qoff, 16)]
            qx2 = qx + qx
            qy2 = qy + qy
            qz2 = qz + qz

            def per_chunk(c, st):
                m1, m2, m3, i1, i2, i3 = st
                off = c * 16
                xc = xs[pl.ds(off, 16)]
                yc = ys[pl.ds(off, 16)]
                zc = zs[pl.ds(off, 16)]
                cbc = cb[pl.ds(off, 16)]
                for k in range(16):
                    d = cbc[k] - (qx2 * xc[k] + qy2 * yc[k] + qz2 * zc[k])
                    iv = jnp.full((16,), off + k, jnp.int32)
                    c1 = d < m1
                    c2 = d < m2
                    c3 = d < m3
                    m3 = jnp.where(c2, m2, jnp.where(c3, d, m3))
                    i3 = jnp.where(c2, i2, jnp.where(c3, iv, i3))
                    m2 = jnp.where(c1, m1, jnp.where(c2, d, m2))
                    i2 = jnp.where(c1, i1, jnp.where(c2, iv, i2))
                    m1 = jnp.where(c1, d, m1)
                    i1 = jnp.where(c1, iv, i1)
                return m1, m2, m3, i1, i2, i3

            init = (jnp.full((16,), _INF, jnp.float32),
                    jnp.full((16,), _INF, jnp.float32),
                    jnp.full((16,), _INF, jnp.float32),
                    jnp.zeros((16,), jnp.int32), jnp.zeros((16,), jnp.int32),
                    jnp.zeros((16,), jnp.int32))
            m1, m2, m3, i1, i2, i3 = lax.fori_loop(0, CH, per_chunk, init)

            s = g * 16
            j1v[pl.ds(s, 16)] = i1.astype(jnp.float32)
            j2v[pl.ds(s, 16)] = i2.astype(jnp.float32)
            j3v[pl.ds(s, 16)] = i3.astype(jnp.float32)
            return carry

        lax.fori_loop(0, QW // 16, per_group, 0)

        pltpu.sync_copy(j1v, oj1.at[pl.ds(base, QW)])
        pltpu.sync_copy(j2v, oj2.at[pl.ds(base, QW)])
        pltpu.sync_copy(j3v, oj3.at[pl.ds(base, QW)])

    return knn


def _tc_epi_body(j1_ref, j2_ref, j3_ref, anum_ref, c0_ref, c1_ref, c2_ref,
                 pos_full_ref, o0_ref, o1_ref, o2_ref):
    i = pl.program_id(0)
    xr = pos_full_ref[pl.ds(S_TC + i * R, R), :]
    pos_tab = pos_full_ref[...]
    iota = jax.lax.broadcasted_iota(jnp.int32, (R, N), 1).astype(jnp.float32)

    is_heavy = anum_ref[...] != 1
    ja = jnp.where(is_heavy, j2_ref[...], j1_ref[...])
    jb = jnp.where(is_heavy, j3_ref[...], j2_ref[...])
    ga = jnp.dot(jnp.where(iota == ja, 1.0, 0.0), pos_tab,
                 preferred_element_type=jnp.float32)
    gb = jnp.dot(jnp.where(iota == jb, 1.0, 0.0), pos_tab,
                 preferred_element_type=jnp.float32)
    _frames_rotate(ga - xr, gb - xr, c0_ref, c1_ref, c2_ref,
                   o0_ref, o1_ref, o2_ref)


@jax.jit
def kernel(coeffs, pos, atomic_numbers):
    heavy = atomic_numbers != 1
    colbase_row = (jnp.sum(pos * pos, axis=1)
                   + jnp.where(heavy, 0.0, jnp.inf)).astype(jnp.float32)
    colbase = colbase_row[None, :]
    posT2 = (2.0 * pos).T
    anum = atomic_numbers[:, None]
    c0 = coeffs[:, :, 0]
    c1 = coeffs[:, :, 1]
    c2 = coeffs[:, :, 2]

    full = lambda *dims: pl.BlockSpec(dims, lambda i: (0,) * len(dims))
    rows = lambda *dims: pl.BlockSpec((R,) + dims, lambda i: (i,) + (0,) * len(dims))

    j1, j2, j3 = _sc_knn_make()(pos[:, 0], pos[:, 1], pos[:, 2], colbase_row)

    t0, t1, t2 = pl.pallas_call(
        _tc_main_body,
        grid=(S_TC // R,),
        in_specs=[
            full(3, N),
            full(1, N),
            rows(1),
            rows(64), rows(64), rows(64),
            full(N, 3),
        ],
        out_specs=[rows(64), rows(64), rows(64)],
        out_shape=[jax.ShapeDtypeStruct((S_TC, 64), jnp.float32)] * 3,
    )(posT2, colbase, anum[:S_TC], c0[:S_TC], c1[:S_TC], c2[:S_TC], pos)

    e0, e1_, e2_ = pl.pallas_call(
        _tc_epi_body,
        grid=(N_SC // R,),
        in_specs=[rows(1), rows(1), rows(1), rows(1),
                  rows(64), rows(64), rows(64), full(N, 3)],
        out_specs=[rows(64), rows(64), rows(64)],
        out_shape=[jax.ShapeDtypeStruct((N_SC, 64), jnp.float32)] * 3,
    )(j1[:, None], j2[:, None], j3[:, None], anum[S_TC:],
      c0[S_TC:], c1[S_TC:], c2[S_TC:], pos)

    o0 = jnp.concatenate([t0, e0], axis=0)
    o1 = jnp.concatenate([t1, e1_], axis=0)
    o2 = jnp.concatenate([t2, e2_], axis=0)
    return jnp.stack([o0, o1, o2], axis=-1)
